# baseline (speedup 1.0000x reference)
import functools

import jax
import jax.numpy as jnp
from jax import lax
from jax.experimental import pallas as pl
from jax.experimental.pallas import tpu as pltpu
from jax.experimental.pallas import tpu_sc as plsc

B, S, D = 4, 8192, 1024
NC, NS = 2, 16
NW = NC * NS
RPW = S // NW
CH = 16
NCH = RPW // CH
VPB = 16


def _sc_add(x2, table):
    mesh = plsc.VectorSubcoreMesh(core_axis_name="c", subcore_axis_name="s")

    @functools.partial(
        pl.kernel,
        mesh=mesh,
        out_type=jax.ShapeDtypeStruct((B * S, D), jnp.float32),
        scratch_types=(
            [pltpu.VMEM((CH, D), jnp.float32)]
            + [pltpu.VMEM((CH, D), jnp.float32) for _ in range(B)]
            + [pltpu.SemaphoreType.DMA for _ in range(1 + 2 * B)]
        ),
    )
    def k(x_hbm, t_hbm, o_hbm, tbuf, xb0, xb1, xb2, xb3,
          tsem, is0, is1, is2, is3, os0, os1, os2, os3):
        xbuf = (xb0, xb1, xb2, xb3)
        isem = (is0, is1, is2, is3)
        osem = (os0, os1, os2, os3)
        wid = lax.axis_index("s") * NC + lax.axis_index("c")
        r0 = wid * RPW

        def chunk_body(c, carry):
            trow = r0 + c * CH
            tin = pltpu.make_async_copy(
                t_hbm.at[pl.ds(trow, CH)], tbuf, tsem)
            tin.start()

            @pl.when(c > 0)
            def _():
                for b in range(B):
                    pltpu.make_async_copy(
                        xbuf[b], o_hbm.at[pl.ds(b * S + trow, CH)], osem[b]
                    ).wait()

            xins = []
            for b in range(B):
                cp = pltpu.make_async_copy(
                    x_hbm.at[pl.ds(b * S + trow, CH)], xbuf[b], isem[b])
                cp.start()
                xins.append(cp)
            tin.wait()
            for b in range(B):
                xins[b].wait()
                buf = xbuf[b]

                def vbody(r, inner):
                    for u in range(D // VPB):
                        sl = pl.ds(u * VPB, VPB)
                        buf[r, sl] = buf[r, sl] + tbuf[r, sl]
                    return inner

                lax.fori_loop(0, CH, vbody, 0)
                pltpu.make_async_copy(
                    buf, o_hbm.at[pl.ds(b * S + trow, CH)], osem[b]).start()
            return carry

        lax.fori_loop(0, NCH, chunk_body, 0)
        trow = r0 + (NCH - 1) * CH
        for b in range(B):
            pltpu.make_async_copy(
                xbuf[b], o_hbm.at[pl.ds(b * S + trow, CH)], osem[b]).wait()

    return k(x2, table)


def kernel(x, pos_table):
    out2 = _sc_add(x.reshape(B * S, D), pos_table)
    return out2.reshape(x.shape)
